# baseline (speedup 1.0000x reference)
import functools

import jax
import jax.numpy as jnp
from jax import lax
from jax.experimental import pallas as pl
from jax.experimental.pallas import tpu as pltpu
from jax.experimental.pallas import tpu_sc as plsc

B = 16384
D = 64
N_NEG = 20
BN = B * N_NEG
VOCAB = 1000000

NC = 2
NS = 16
NW = NC * NS

IW = 128
KS = 4
SUP = IW * KS

G1_ROWS = B // NW
G3_ROWS = BN // NW
G3_SUPER = G3_ROWS // SUP

TBLK = 8192
NBLK = 63
Q = NBLK * TBLK
SPLIT = 62 * TBLK
TAIL_BLK = VOCAB // TBLK
TAIL_OFF = TAIL_BLK * TBLK - SPLIT


def _tc_relayout_body(xa_ref, xb_ref, o_ref):
    o_ref[:, 0:D] = xa_ref[...].T
    o_ref[:, D:2 * D] = xb_ref[...].T


def _tc_relayout(table_t):
    return pl.pallas_call(
        _tc_relayout_body,
        grid=(NBLK,),
        in_specs=[
            pl.BlockSpec((D, TBLK),
                         lambda i: (0, jnp.where(i == NBLK - 1, TAIL_BLK, i))),
            pl.BlockSpec((D, TBLK),
                         lambda i: (0, jnp.minimum(i + 62, TAIL_BLK))),
        ],
        out_specs=pl.BlockSpec((TBLK, 2 * D), lambda i: (i, 0)),
        out_shape=jax.ShapeDtypeStruct((Q, 2 * D), jnp.float32),
    )(table_t, table_t)


def _remap(r):
    return jnp.where(
        r < SPLIT,
        2 * r,
        jnp.where(r < TAIL_BLK * TBLK,
                  2 * (r - SPLIT) + 1,
                  2 * (r - TAIL_OFF)))


def _build_kernel():
    mesh = plsc.VectorSubcoreMesh(core_axis_name="c", subcore_axis_name="s")

    @functools.partial(
        pl.kernel,
        mesh=mesh,
        compiler_params=pltpu.CompilerParams(use_tc_tiling_on_sc=False),
        out_type=[
            jax.ShapeDtypeStruct((B, D), jnp.float32),
            jax.ShapeDtypeStruct((B, D), jnp.float32),
            jax.ShapeDtypeStruct((BN, D), jnp.float32),
        ],
        scratch_types=[
            pltpu.VMEM((KS, IW), jnp.int32),
            pltpu.VMEM((SUP, D), jnp.float32),
            pltpu.SemaphoreType.DMA,
        ],
    )
    def k(in_tok, ctx_tok, neg_tok, in_tab, out_tab, o1, o2, o3,
          idx_v, rows_v, gsem):
        wid = lax.axis_index("s") * NC + lax.axis_index("c")

        def one_super(tok2d, tab, out, r0, o0):
            pltpu.sync_copy(tok2d.at[pl.ds(r0, KS)], idx_v)
            cps = [
                pltpu.async_copy(
                    tab.at[idx_v.at[j]], rows_v.at[pl.ds(j * IW, IW)], gsem)
                for j in range(KS)
            ]
            for c in cps:
                c.wait()
            pltpu.sync_copy(rows_v, out.at[pl.ds(o0, SUP)])

        one_super(in_tok, in_tab, o1, wid * KS, wid * SUP)
        one_super(ctx_tok, out_tab, o2, wid * KS, wid * SUP)

        def body(s, carry):
            one_super(neg_tok, out_tab, o3,
                      wid * (G3_ROWS // IW) + s * KS,
                      wid * G3_ROWS + s * SUP)
            return carry

        lax.fori_loop(0, G3_SUPER, body, 0)

    return k


_gather_kernel = _build_kernel()


@jax.jit
def kernel(input_tokens, context_tokens, negative_context, in_table, out_table):
    in2d = input_tokens.reshape(B // IW, IW)
    ctx2d = _remap(context_tokens).reshape(B // IW, IW)
    neg2d = _remap(negative_context).reshape(BN // IW, IW)
    comb = _tc_relayout(out_table.T).reshape(2 * Q, D)
    o1, o2, o3 = _gather_kernel(in2d, ctx2d, neg2d, in_table, comb)
    return o1, o2, o3.reshape(B, N_NEG, D)


# speedup vs baseline: 1.4319x; 1.4319x over previous
import functools

import jax
import jax.numpy as jnp
from jax import lax
from jax.experimental import pallas as pl
from jax.experimental.pallas import tpu as pltpu
from jax.experimental.pallas import tpu_sc as plsc

B = 16384
D = 64
N_NEG = 20
BN = B * N_NEG
VOCAB = 1000000

NC = 2
NS = 16
NW = NC * NS

IW = 128
KS = 4
SUP = IW * KS

G1_ROWS = B // NW
G3_ROWS = BN // NW
G3_SUPER = G3_ROWS // SUP

TBLK = 8192
NBLK = 63
Q = NBLK * TBLK
SPLIT = 62 * TBLK
TAIL_BLK = VOCAB // TBLK
TAIL_OFF = TAIL_BLK * TBLK - SPLIT


def _tc_relayout_body(xa_ref, xb_ref, o_ref):
    o_ref[:, 0:D] = xa_ref[...].T
    o_ref[:, D:2 * D] = xb_ref[...].T


def _tc_relayout(table_t):
    return pl.pallas_call(
        _tc_relayout_body,
        grid=(NBLK,),
        in_specs=[
            pl.BlockSpec((D, TBLK),
                         lambda i: (0, jnp.where(i == NBLK - 1, TAIL_BLK, i))),
            pl.BlockSpec((D, TBLK),
                         lambda i: (0, jnp.minimum(i + 62, TAIL_BLK))),
        ],
        out_specs=pl.BlockSpec((TBLK, 2 * D), lambda i: (i, 0)),
        out_shape=jax.ShapeDtypeStruct((Q, 2 * D), jnp.float32),
    )(table_t, table_t)


def _remap(r):
    return jnp.where(
        r < SPLIT,
        2 * r,
        jnp.where(r < TAIL_BLK * TBLK,
                  2 * (r - SPLIT) + 1,
                  2 * (r - TAIL_OFF)))


def _one_super(tab, idx_v, rows_v, gsem, tok2d, out, r0, o0):
    pltpu.sync_copy(tok2d.at[pl.ds(r0, KS)], idx_v)
    cps = [
        pltpu.async_copy(
            tab.at[idx_v.at[j]], rows_v.at[pl.ds(j * IW, IW)], gsem)
        for j in range(KS)
    ]
    for c in cps:
        c.wait()
    pltpu.sync_copy(rows_v, out.at[pl.ds(o0, SUP)])


_SCRATCH = [
    pltpu.VMEM((KS, IW), jnp.int32),
    pltpu.VMEM((SUP, D), jnp.float32),
    pltpu.SemaphoreType.DMA,
]
_MESH = plsc.VectorSubcoreMesh(core_axis_name="c", subcore_axis_name="s")
_SC_PARAMS = pltpu.CompilerParams(use_tc_tiling_on_sc=False)


@functools.partial(
    pl.kernel,
    mesh=_MESH,
    compiler_params=_SC_PARAMS,
    out_type=[
        jax.ShapeDtypeStruct((B, D), jnp.float32),
        jax.ShapeDtypeStruct((BN, D), jnp.float32),
    ],
    scratch_types=_SCRATCH,
)
def _gather_ctx_neg(ctx_tok, neg_tok, tab, o2, o3, idx_v, rows_v, gsem):
    wid = lax.axis_index("s") * NC + lax.axis_index("c")
    _one_super(tab, idx_v, rows_v, gsem, ctx_tok, o2, wid * KS, wid * SUP)

    def body(s, carry):
        _one_super(tab, idx_v, rows_v, gsem, neg_tok, o3,
                   wid * (G3_ROWS // IW) + s * KS,
                   wid * G3_ROWS + s * SUP)
        return carry

    lax.fori_loop(0, G3_SUPER, body, 0)


@functools.partial(
    pl.kernel,
    mesh=_MESH,
    compiler_params=_SC_PARAMS,
    out_type=jax.ShapeDtypeStruct((B, D), jnp.float32),
    scratch_types=_SCRATCH,
)
def _gather_in(in_tok, tab, o1, idx_v, rows_v, gsem):
    wid = lax.axis_index("s") * NC + lax.axis_index("c")
    _one_super(tab, idx_v, rows_v, gsem, in_tok, o1, wid * KS, wid * SUP)


@jax.jit
def kernel(input_tokens, context_tokens, negative_context, in_table, out_table):
    in2d = _remap(input_tokens).reshape(B // IW, IW)
    ctx2d = _remap(context_tokens).reshape(B // IW, IW)
    neg2d = _remap(negative_context).reshape(BN // IW, IW)
    comb_out = _tc_relayout(out_table.T).reshape(2 * Q, D)
    o2, o3 = _gather_ctx_neg(ctx2d, neg2d, comb_out)
    comb_in = _tc_relayout(in_table.T).reshape(2 * Q, D)
    o1 = _gather_in(in2d, comb_in)
    return o1, o2, o3.reshape(B, N_NEG, D)


# speedup vs baseline: 1.4967x; 1.0452x over previous
import functools

import jax
import jax.numpy as jnp
from jax import lax
from jax.experimental import pallas as pl
from jax.experimental.pallas import tpu as pltpu
from jax.experimental.pallas import tpu_sc as plsc

B = 16384
D = 64
N_NEG = 20
BN = B * N_NEG
VOCAB = 1000000

NC = 2
NS = 16
NW = NC * NS

IW = 128
KS = 4
SUP = IW * KS

G1_ROWS = B // NW
G3_ROWS = BN // NW
G3_SUPER = G3_ROWS // SUP

TBLK = 16384
NBLK = 31
Q = NBLK * TBLK
SPLIT = 30 * TBLK
TAIL_BLK = VOCAB // TBLK
TAIL_OFF = TAIL_BLK * TBLK - SPLIT


def _tc_relayout_body(xa_ref, xb_ref, o_ref):
    o_ref[:, 0:D] = xa_ref[...].T
    o_ref[:, D:2 * D] = xb_ref[...].T


def _tc_relayout(table_t):
    return pl.pallas_call(
        _tc_relayout_body,
        grid=(NBLK,),
        in_specs=[
            pl.BlockSpec((D, TBLK),
                         lambda i: (0, jnp.where(i == NBLK - 1, TAIL_BLK, i))),
            pl.BlockSpec((D, TBLK),
                         lambda i: (0, jnp.minimum(i + NBLK - 1, TAIL_BLK - 1))),
        ],
        out_specs=pl.BlockSpec((TBLK, 2 * D), lambda i: (i, 0)),
        out_shape=jax.ShapeDtypeStruct((Q, 2 * D), jnp.float32),
    )(table_t, table_t)


def _remap(r):
    return jnp.where(
        r < SPLIT,
        2 * r,
        jnp.where(r < TAIL_BLK * TBLK,
                  2 * (r - SPLIT) + 1,
                  2 * (r - TAIL_OFF)))


def _one_super(tab, idx_v, rows_v, gsem, tok2d, out, r0, o0):
    pltpu.sync_copy(tok2d.at[pl.ds(r0, KS)], idx_v)
    cps = [
        pltpu.async_copy(
            tab.at[idx_v.at[j]], rows_v.at[pl.ds(j * IW, IW)], gsem)
        for j in range(KS)
    ]
    for c in cps:
        c.wait()
    pltpu.sync_copy(rows_v, out.at[pl.ds(o0, SUP)])


_SCRATCH = [
    pltpu.VMEM((KS, IW), jnp.int32),
    pltpu.VMEM((SUP, D), jnp.float32),
    pltpu.SemaphoreType.DMA,
]
_MESH = plsc.VectorSubcoreMesh(core_axis_name="c", subcore_axis_name="s")
_SC_PARAMS = pltpu.CompilerParams(use_tc_tiling_on_sc=False)


@functools.partial(
    pl.kernel,
    mesh=_MESH,
    compiler_params=_SC_PARAMS,
    out_type=[
        jax.ShapeDtypeStruct((B, D), jnp.float32),
        jax.ShapeDtypeStruct((BN, D), jnp.float32),
    ],
    scratch_types=_SCRATCH,
)
def _gather_ctx_neg(ctx_tok, neg_tok, tab, o2, o3, idx_v, rows_v, gsem):
    wid = lax.axis_index("s") * NC + lax.axis_index("c")
    _one_super(tab, idx_v, rows_v, gsem, ctx_tok, o2, wid * KS, wid * SUP)

    def body(s, carry):
        _one_super(tab, idx_v, rows_v, gsem, neg_tok, o3,
                   wid * (G3_ROWS // IW) + s * KS,
                   wid * G3_ROWS + s * SUP)
        return carry

    lax.fori_loop(0, G3_SUPER, body, 0)


@functools.partial(
    pl.kernel,
    mesh=_MESH,
    compiler_params=_SC_PARAMS,
    out_type=jax.ShapeDtypeStruct((B, D), jnp.float32),
    scratch_types=_SCRATCH,
)
def _gather_in(in_tok, tab, o1, idx_v, rows_v, gsem):
    wid = lax.axis_index("s") * NC + lax.axis_index("c")
    _one_super(tab, idx_v, rows_v, gsem, in_tok, o1, wid * KS, wid * SUP)


@jax.jit
def kernel(input_tokens, context_tokens, negative_context, in_table, out_table):
    in2d = _remap(input_tokens).reshape(B // IW, IW)
    ctx2d = _remap(context_tokens).reshape(B // IW, IW)
    neg2d = _remap(negative_context).reshape(BN // IW, IW)
    comb_out = _tc_relayout(out_table.T).reshape(2 * Q, D)
    o2, o3 = _gather_ctx_neg(ctx2d, neg2d, comb_out)
    comb_in = _tc_relayout(in_table.T).reshape(2 * Q, D)
    o1 = _gather_in(in2d, comb_in)
    return o1, o2, o3.reshape(B, N_NEG, D)
